# baseline (speedup 1.0000x reference)
import jax
import jax.numpy as jnp
from jax.experimental import pallas as pl
from jax.experimental.pallas import tpu as pltpu

_MAX_TILES = 4
_HIDDEN = 1280
_PATCHES = 1025


def _body(ids_ref, gate_ref, hid_ref, emb_ref, out_ref):
    g = jnp.tanh(gate_ref[0])
    out_ref[...] = hid_ref[...] + emb_ref[...] * g


def kernel(hidden_state, aspect_ratio_ids, embedding_table, gate):
    batch = hidden_state.shape[0]
    ids = aspect_ratio_ids.astype(jnp.int32)
    table = embedding_table.reshape(-1, _MAX_TILES, 1, _HIDDEN)
    grid = (batch, _MAX_TILES)

    out = pl.pallas_call(
        _body,
        grid_spec=pltpu.PrefetchScalarGridSpec(
            num_scalar_prefetch=2,
            grid=grid,
            in_specs=[
                pl.BlockSpec(
                    (1, 1, _PATCHES, _HIDDEN),
                    lambda b, t, ids, gate: (b, t, 0, 0),
                ),
                pl.BlockSpec(
                    (1, 1, 1, _HIDDEN),
                    lambda b, t, ids, gate: (ids[b], t, 0, 0),
                ),
            ],
            out_specs=pl.BlockSpec(
                (1, 1, _PATCHES, _HIDDEN),
                lambda b, t, ids, gate: (b, t, 0, 0),
            ),
        ),
        out_shape=jax.ShapeDtypeStruct(hidden_state.shape, hidden_state.dtype),
        compiler_params=pltpu.CompilerParams(
            dimension_semantics=("parallel", "parallel"),
        ),
    )(ids, gate, hidden_state, table)
    return out
